# Initial kernel scaffold; baseline (speedup 1.0000x reference)
import jax
import jax.numpy as jnp
from jax.experimental import pallas as pl


def kernel(mm_embed, task_index, true_y, gate_W, gate_b, exp_W1, exp_b1, exp_W2, exp_b2, ln_g, ln_b, head_W, head_b):
    raise NotImplementedError("write your pallas kernel here")


# speedup vs baseline: 6.0052x; 6.0052x over previous
import functools

import jax
import jax.numpy as jnp
from jax.experimental import pallas as pl
from jax.experimental.pallas import tpu as pltpu

_B, _S, _D = 1, 2048, 768
_E, _T, _K, _H = 16, 5, 2, 768
_N = _B * _S


def _cv2(v, n):
    mu = jnp.sum(v, keepdims=True) / n
    var = jnp.sum((v - mu) ** 2, keepdims=True) / (n - 1)
    return var / (mu * mu + 1e-10)


def _moe_body(x_ref, gwt_ref, gb_ref, ti_ref, ty_ref, w1_ref, b1_ref,
              w2_ref, b2_ref, lng_ref, lnb_ref, hw_ref, hb_ref,
              scores_ref, tot_ref, ploss_ref,
              gsel_ref, ysum_ref, loss_ref, imp_ref):
    e = pl.program_id(0)
    io = jax.lax.broadcasted_iota(jnp.int32, (_N, _E), 1)

    @pl.when(e == 0)
    def _gating():
        x = x_ref[...]
        logits = jax.lax.dot_general(
            x, gwt_ref[...], (((1,), (1,)), ((), ())),
            preferred_element_type=jnp.float32) + gb_ref[...]
        ti = ti_ref[...]
        loss = jnp.zeros((1, 1), jnp.float32)
        gacc = jnp.zeros((_N, _E), jnp.float32)
        for t in range(_T):
            lt = logits[:, t * _E:(t + 1) * _E]
            m1 = jnp.max(lt, axis=1, keepdims=True)
            i1 = jnp.min(jnp.where(lt == m1, io, _E), axis=1, keepdims=True)
            lt2 = jnp.where(io == i1, -jnp.inf, lt)
            m2 = jnp.max(lt2, axis=1, keepdims=True)
            i2 = jnp.min(jnp.where(lt2 == m2, io, _E), axis=1, keepdims=True)
            wa = jax.nn.sigmoid(m1 - m2)
            wb = jax.nn.sigmoid(m2 - m1)
            gt = (jnp.where(io == i1, wa, 0.0)
                  + jnp.where(io == i2, wb, 0.0))
            imp = jnp.sum(gt, axis=0, keepdims=True)
            ld = jnp.sum((gt > 0).astype(jnp.float32), axis=0, keepdims=True)
            loss = loss + _cv2(imp, _E) + _cv2(ld, _E)
            gacc = gacc + jnp.where(ti == t, gt, 0.0)
        gsel_ref[...] = gacc
        loss_ref[...] = loss
        imp_ref[...] = jnp.sum(gacc, axis=0, keepdims=True)
        ysum_ref[...] = jnp.zeros((1, _H), jnp.float32)

    x = x_ref[...]
    h = jax.nn.relu(
        jax.lax.dot_general(x, w1_ref[0], (((1,), (1,)), ((), ())),
                            preferred_element_type=jnp.float32)
        + b1_ref[0])
    gcol = jnp.sum(jnp.where(io == e, gsel_ref[...], 0.0),
                   axis=1, keepdims=True)
    v = jnp.sum(h * gcol, axis=0, keepdims=True)
    imp_e = jnp.sum(jnp.where(
        jax.lax.broadcasted_iota(jnp.int32, (1, _E), 1) == e,
        imp_ref[...], 0.0), axis=1, keepdims=True)
    ydelta = jax.lax.dot_general(
        v, w2_ref[0], (((1,), (1,)), ((), ())),
        preferred_element_type=jnp.float32) + imp_e * b2_ref[0]
    ysum_ref[...] += ydelta

    @pl.when(e == _E - 1)
    def _final():
        mm = ysum_ref[...]
        mu = jnp.sum(mm, keepdims=True) / _H
        var = jnp.sum((mm - mu) ** 2, keepdims=True) / _H
        fin = (mm - mu) / jnp.sqrt(var + 1e-5) * lng_ref[...] + lnb_ref[...]
        out = jnp.sum(fin * hw_ref[...], keepdims=True) + hb_ref[...]
        sc = jax.nn.sigmoid(out)
        scores_ref[...] = sc
        tot_ref[...] = loss_ref[...] * 0.01
        ploss_ref[...] = (sc - ty_ref[...]) ** 2


@functools.partial(jax.jit, static_argnums=())
def kernel(mm_embed, task_index, true_y, gate_W, gate_b, exp_W1, exp_b1,
           exp_W2, exp_b2, ln_g, ln_b, head_W, head_b):
    x = mm_embed.reshape(_N, _D)
    gwt = gate_W.reshape(_T * _E, _D)
    gb = gate_b.reshape(1, _T * _E)
    ti = task_index.reshape(1, 1)
    ty = true_y.reshape(1, 1)

    full = lambda s: pl.BlockSpec(s, lambda e: (0,) * len(s))
    per_eb = pl.BlockSpec((1, 1, _D), lambda e: (e, 0, 0))
    per_e3 = pl.BlockSpec((1, _H, _D), lambda e: (e, 0, 0))
    per_e3b = pl.BlockSpec((1, _D, _H), lambda e: (e, 0, 0))

    scores, tot, ploss = pl.pallas_call(
        _moe_body,
        grid=(_E,),
        in_specs=[
            full((_N, _D)),
            full((_T * _E, _D)),
            full((1, _T * _E)),
            full((1, 1)),
            full((1, 1)),
            per_e3,
            per_eb,
            per_e3b,
            per_eb,
            full((1, _H)),
            full((1, _H)),
            full((1, _H)),
            full((1, 1)),
        ],
        out_specs=[full((1, 1)), full((1, 1)), full((1, 1))],
        out_shape=[jax.ShapeDtypeStruct((1, 1), jnp.float32)] * 3,
        scratch_shapes=[
            pltpu.VMEM((_N, _E), jnp.float32),
            pltpu.VMEM((1, _H), jnp.float32),
            pltpu.VMEM((1, 1), jnp.float32),
            pltpu.VMEM((1, _E), jnp.float32),
        ],
    )(x, gwt, gb, ti, ty, exp_W1, exp_b1.reshape(_E, 1, _H),
      exp_W2, exp_b2.reshape(_E, 1, _D),
      ln_g.reshape(1, _H), ln_b.reshape(1, _H), head_W.reshape(1, _H),
      head_b.reshape(1, 1))

    return (scores.astype(jnp.float32),
            tot.reshape(()).astype(jnp.float32),
            ploss.reshape(()).astype(jnp.float32))
